# baseline (speedup 1.0000x reference)
import functools

import jax
import jax.numpy as jnp
from jax import lax
from jax.experimental import pallas as pl
from jax.experimental.pallas import tpu as pltpu
from jax.experimental.pallas import tpu_sc as plsc

B = 16384
WIN = 48
WOUT = 63
NWORK = 32
RPW = B // NWORK
CHR = 256
OFFS = (0, 16, 32, 47)


def _sc_body(cm_ref, x_hbm, o_hbm, in_v, out_v, cm_v):
    wid = lax.axis_index("s") * 2 + lax.axis_index("c")
    pltpu.sync_copy(cm_ref, cm_v)
    cols = [cm_v[pl.ds(k * 16, 16)] for k in range(4)]

    def row(r, _):
        rv = jnp.full((16,), r, jnp.int32)
        for k in range(4):
            out_v[r, pl.ds(OFFS[k], 16)] = plsc.load_gather(
                in_v, [rv, cols[k]])
        return _

    for c in range(RPW // CHR):
        base = wid * RPW + c * CHR
        pltpu.sync_copy(x_hbm.at[pl.ds(base, CHR), :], in_v)
        lax.fori_loop(0, CHR, row, None)
        pltpu.sync_copy(out_v, o_hbm.at[pl.ds(base, CHR), :])


def _sc_call(in2d, cm):
    f = functools.partial(
        pl.kernel,
        out_type=jax.ShapeDtypeStruct((B, WOUT), jnp.float32),
        mesh=plsc.VectorSubcoreMesh(core_axis_name="c", subcore_axis_name="s"),
        scratch_types=[
            pltpu.VMEM((CHR, WIN), jnp.float32),
            pltpu.VMEM((CHR, WOUT), jnp.float32),
            pltpu.VMEM((64,), jnp.int32),
        ],
        compiler_params=pltpu.CompilerParams(
            needs_layout_passes=False, use_tc_tiling_on_sc=True),
    )(_sc_body)
    return f(cm, in2d)


def kernel(joints, indices):
    cmap = (3 * jnp.repeat(indices.astype(jnp.int32), 3)
            + jnp.tile(jnp.arange(3, dtype=jnp.int32), 21))
    cm = jnp.concatenate(
        [cmap[o:o + 16] for o in OFFS]).astype(jnp.int32)
    out2d = _sc_call(joints.reshape(B, WIN), cm)
    return out2d.reshape(B, 21, 3)
